# baseline (speedup 1.0000x reference)
import functools

import jax
import jax.numpy as jnp
from jax import lax
from jax.experimental import pallas as pl
from jax.experimental.pallas import tpu as pltpu
from jax.experimental.pallas import tpu_sc as plsc

_VOCAB = 100000
_SEQ = 2048
_EMBED = 128
_BATCH = 32

_NC = 2
_NS = 16
_NW = _NC * _NS
_CHUNK = 128


def _sc_gather(ids_part, table):
    nrow = ids_part.shape[0]
    per_w = nrow * _SEQ // _NW
    wpr = _NW // nrow
    nch = per_w // _CHUNK
    mesh = plsc.VectorSubcoreMesh(core_axis_name="c", subcore_axis_name="s")

    @functools.partial(
        pl.kernel,
        mesh=mesh,
        out_type=jax.ShapeDtypeStruct((nrow, _SEQ, _EMBED), jnp.float32),
        scratch_types=[
            pltpu.VMEM((per_w,), jnp.int32),
            pltpu.VMEM((_CHUNK, _EMBED), jnp.float32),
            pltpu.VMEM((_CHUNK, _EMBED), jnp.float32),
            pltpu.VMEM((_CHUNK, _EMBED), jnp.float32),
            pltpu.SemaphoreType.DMA,
            pltpu.SemaphoreType.DMA,
            pltpu.SemaphoreType.DMA,
            pltpu.SemaphoreType.DMA,
            pltpu.SemaphoreType.DMA,
            pltpu.SemaphoreType.DMA,
        ],
    )
    def k(ids_hbm, table_hbm, out_hbm, idx_v, rows0, rows1, rows2,
          g0, g1, g2s, o0, o1, o2):
        wid = lax.axis_index("s") * _NC + lax.axis_index("c")
        row = wid // wpr
        base = (wid % wpr) * per_w
        rows = (rows0, rows1, rows2)
        gsem = (g0, g1, g2s)
        osem = (o0, o1, o2)
        pltpu.sync_copy(ids_hbm.at[row, pl.ds(base, per_w)], idx_v)

        def gather(c):
            b = c % 3
            pltpu.async_copy(
                table_hbm.at[idx_v.at[pl.ds(c * _CHUNK, _CHUNK)]],
                rows[b], gsem[b])

        def writeout(c):
            b = c % 3
            pltpu.async_copy(
                rows[b], out_hbm.at[row, pl.ds(base + c * _CHUNK, _CHUNK)],
                osem[b])

        def wait_gather(b):
            pltpu.make_async_copy(table_hbm.at[idx_v.at[pl.ds(0, _CHUNK)]],
                                  rows[b], gsem[b]).wait()

        def wait_writeout(b):
            pltpu.make_async_copy(
                rows[b], out_hbm.at[row, pl.ds(0, _CHUNK)], osem[b]).wait()

        gather(0)
        if nch > 1:
            gather(1)
        for c in range(nch):
            b = c % 3
            wait_gather(b)
            if c + 2 < nch:
                b2 = (c + 2) % 3
                if c >= 1:
                    wait_writeout(b2)
                gather(c + 2)
            writeout(c)
        for c in range(max(0, nch - 3), nch):
            wait_writeout(c % 3)

    return k(ids_part, table)


_PANEL = 128


def _make_tc_body(nrow, row_off, with_prev):
    def body(*refs):
        if with_prev:
            (tok_ref, segt_ref, pos_ref, st_ref, g_ref, b_ref,
             _prev_ref, out_ref) = refs
        else:
            (tok_ref, segt_ref, pos_ref, st_ref, g_ref, b_ref,
             out_ref) = refs
        ib = pl.program_id(0) + row_off
        s0 = st_ref[0:1, :]
        d = st_ref[1:2, :] - s0
        a = jnp.full((_EMBED, _EMBED), 1.0 / _EMBED, dtype=jnp.bfloat16)
        g = g_ref[...]
        b = b_ref[...]
        lane = lax.broadcasted_iota(jnp.int32, (1, _BATCH), 1)
        oneh = (lane == ib).astype(jnp.float32)
        for i in range(_SEQ // _PANEL):
            sl = pl.ds(i * _PANEL, _PANEL)
            segc = jnp.sum(segt_ref[sl, :] * oneh, axis=1, keepdims=True)
            x = tok_ref[0, sl, :] + pos_ref[sl, :] + (s0 + segc * d)
            xb = x.astype(jnp.bfloat16)
            x2b = xb * xb
            mu = jnp.dot(xb, a, preferred_element_type=jnp.float32)
            msq = jnp.dot(x2b, a, preferred_element_type=jnp.float32)
            rstd = lax.rsqrt(msq - mu * mu + 1e-5)
            out_ref[0, sl, :] = (x - mu) * (rstd * g) + b
    return body


def _tc_part(tok_part, seg_t, position_table, segment_table, g2, b2,
             row_off, prev):
    nrow = tok_part.shape[0]
    in_specs = [
        pl.BlockSpec((1, _SEQ, _EMBED), lambda i: (i, 0, 0)),
        pl.BlockSpec((_SEQ, _BATCH), lambda i: (0, 0)),
        pl.BlockSpec((_SEQ, _EMBED), lambda i: (0, 0)),
        pl.BlockSpec((2, _EMBED), lambda i: (0, 0)),
        pl.BlockSpec((1, _EMBED), lambda i: (0, 0)),
        pl.BlockSpec((1, _EMBED), lambda i: (0, 0)),
    ]
    args = [tok_part, seg_t, position_table, segment_table, g2, b2]
    aliases = {}
    if prev is not None:
        in_specs.append(pl.BlockSpec(memory_space=pl.ANY))
        args.append(prev)
        aliases = {6: 0}
    return pl.pallas_call(
        _make_tc_body(nrow, row_off, prev is not None),
        grid=(nrow,),
        in_specs=in_specs,
        out_specs=pl.BlockSpec((1, _SEQ, _EMBED),
                               lambda i: (i + row_off, 0, 0)),
        out_shape=jax.ShapeDtypeStruct((_BATCH, _SEQ, _EMBED), jnp.float32),
        input_output_aliases=aliases,
    )(*args)


_NSPLIT = 2


def kernel(input_ids, segment_ids, token_table, position_table, segment_table,
           ln_gamma, ln_beta):
    ids = input_ids.astype(jnp.int32)
    seg_t = segment_ids.astype(jnp.float32).T
    g2 = jnp.reshape(ln_gamma, (1, _EMBED))
    b2 = jnp.reshape(ln_beta, (1, _EMBED))
    h = _BATCH // _NSPLIT
    toks = [_sc_gather(ids[i * h:(i + 1) * h], token_table)
            for i in range(_NSPLIT)]
    out = None
    for i in range(_NSPLIT):
        out = _tc_part(toks[i], seg_t, position_table, segment_table,
                       g2, b2, i * h, out)
    return out


# speedup vs baseline: 1.0193x; 1.0193x over previous
import functools

import jax
import jax.numpy as jnp
from jax import lax
from jax.experimental import pallas as pl
from jax.experimental.pallas import tpu as pltpu
from jax.experimental.pallas import tpu_sc as plsc

_VOCAB = 100000
_SEQ = 2048
_EMBED = 128
_BATCH = 32

_NC = 2
_NS = 16
_NW = _NC * _NS
_CHUNK = 256


def _sc_gather(ids_part, table):
    nrow = ids_part.shape[0]
    per_w = nrow * _SEQ // _NW
    wpr = _NW // nrow
    nch = per_w // _CHUNK
    mesh = plsc.VectorSubcoreMesh(core_axis_name="c", subcore_axis_name="s")

    @functools.partial(
        pl.kernel,
        mesh=mesh,
        out_type=jax.ShapeDtypeStruct((nrow, _SEQ, _EMBED), jnp.float32),
        scratch_types=[
            pltpu.VMEM((per_w,), jnp.int32),
            pltpu.VMEM((_CHUNK, _EMBED), jnp.float32),
            pltpu.VMEM((_CHUNK, _EMBED), jnp.float32),
            pltpu.VMEM((_CHUNK, _EMBED), jnp.float32),
            pltpu.SemaphoreType.DMA,
            pltpu.SemaphoreType.DMA,
            pltpu.SemaphoreType.DMA,
            pltpu.SemaphoreType.DMA,
            pltpu.SemaphoreType.DMA,
            pltpu.SemaphoreType.DMA,
        ],
    )
    def k(ids_hbm, table_hbm, out_hbm, idx_v, rows0, rows1, rows2,
          g0, g1, g2s, o0, o1, o2):
        wid = lax.axis_index("s") * _NC + lax.axis_index("c")
        row = wid // wpr
        base = (wid % wpr) * per_w
        rows = (rows0, rows1, rows2)
        gsem = (g0, g1, g2s)
        osem = (o0, o1, o2)
        pltpu.sync_copy(ids_hbm.at[row, pl.ds(base, per_w)], idx_v)

        def gather(c):
            b = c % 3
            pltpu.async_copy(
                table_hbm.at[idx_v.at[pl.ds(c * _CHUNK, _CHUNK)]],
                rows[b], gsem[b])

        def writeout(c):
            b = c % 3
            pltpu.async_copy(
                rows[b], out_hbm.at[row, pl.ds(base + c * _CHUNK, _CHUNK)],
                osem[b])

        def wait_gather(b):
            pltpu.make_async_copy(table_hbm.at[idx_v.at[pl.ds(0, _CHUNK)]],
                                  rows[b], gsem[b]).wait()

        def wait_writeout(b):
            pltpu.make_async_copy(
                rows[b], out_hbm.at[row, pl.ds(0, _CHUNK)], osem[b]).wait()

        gather(0)
        if nch > 1:
            gather(1)
        for c in range(nch):
            b = c % 3
            wait_gather(b)
            if c + 2 < nch:
                b2 = (c + 2) % 3
                if c >= 1:
                    wait_writeout(b2)
                gather(c + 2)
            writeout(c)
        for c in range(max(0, nch - 3), nch):
            wait_writeout(c % 3)

    return k(ids_part, table)


_PANEL = 128


def _make_tc_body(nrow, row_off, with_prev):
    def body(*refs):
        if with_prev:
            (tok_ref, segt_ref, pos_ref, st_ref, g_ref, b_ref,
             _prev_ref, out_ref) = refs
        else:
            (tok_ref, segt_ref, pos_ref, st_ref, g_ref, b_ref,
             out_ref) = refs
        ib = pl.program_id(0) + row_off
        s0 = st_ref[0:1, :]
        d = st_ref[1:2, :] - s0
        a = jnp.full((_EMBED, _EMBED), 1.0 / _EMBED, dtype=jnp.bfloat16)
        g = g_ref[...]
        b = b_ref[...]
        lane = lax.broadcasted_iota(jnp.int32, (1, _BATCH), 1)
        oneh = (lane == ib).astype(jnp.float32)
        for i in range(_SEQ // _PANEL):
            sl = pl.ds(i * _PANEL, _PANEL)
            segc = jnp.sum(segt_ref[sl, :] * oneh, axis=1, keepdims=True)
            x = tok_ref[0, sl, :] + pos_ref[sl, :] + (s0 + segc * d)
            xb = x.astype(jnp.bfloat16)
            x2b = xb * xb
            mu = jnp.dot(xb, a, preferred_element_type=jnp.float32)
            msq = jnp.dot(x2b, a, preferred_element_type=jnp.float32)
            rstd = lax.rsqrt(msq - mu * mu + 1e-5)
            out_ref[0, sl, :] = (x - mu) * (rstd * g) + b
    return body


def _tc_part(tok_part, seg_t, position_table, segment_table, g2, b2,
             row_off, prev):
    nrow = tok_part.shape[0]
    in_specs = [
        pl.BlockSpec((1, _SEQ, _EMBED), lambda i: (i, 0, 0)),
        pl.BlockSpec((_SEQ, _BATCH), lambda i: (0, 0)),
        pl.BlockSpec((_SEQ, _EMBED), lambda i: (0, 0)),
        pl.BlockSpec((2, _EMBED), lambda i: (0, 0)),
        pl.BlockSpec((1, _EMBED), lambda i: (0, 0)),
        pl.BlockSpec((1, _EMBED), lambda i: (0, 0)),
    ]
    args = [tok_part, seg_t, position_table, segment_table, g2, b2]
    aliases = {}
    if prev is not None:
        in_specs.append(pl.BlockSpec(memory_space=pl.ANY))
        args.append(prev)
        aliases = {6: 0}
    return pl.pallas_call(
        _make_tc_body(nrow, row_off, prev is not None),
        grid=(nrow,),
        in_specs=in_specs,
        out_specs=pl.BlockSpec((1, _SEQ, _EMBED),
                               lambda i: (i + row_off, 0, 0)),
        out_shape=jax.ShapeDtypeStruct((_BATCH, _SEQ, _EMBED), jnp.float32),
        input_output_aliases=aliases,
    )(*args)


_NSPLIT = 2


def kernel(input_ids, segment_ids, token_table, position_table, segment_table,
           ln_gamma, ln_beta):
    ids = input_ids.astype(jnp.int32)
    seg_t = segment_ids.astype(jnp.float32).T
    g2 = jnp.reshape(ln_gamma, (1, _EMBED))
    b2 = jnp.reshape(ln_beta, (1, _EMBED))
    h = _BATCH // _NSPLIT
    toks = [_sc_gather(ids[i * h:(i + 1) * h], token_table)
            for i in range(_NSPLIT)]
    out = None
    for i in range(_NSPLIT):
        out = _tc_part(toks[i], seg_t, position_table, segment_table,
                       g2, b2, i * h, out)
    return out
